# baseline (speedup 1.0000x reference)
import functools

import jax
import jax.numpy as jnp
from jax import lax
from jax.experimental import pallas as pl
from jax.experimental.pallas import tpu as pltpu
from jax.experimental.pallas import tpu_sc as plsc

D = 32
L = 16
NC = 2
NS = 16
NW = NC * NS
CH = 128


def _dot_body(b_per_w, sid_hbm, pid_hbm, semb_hbm, pemb_hbm, out_hbm,
              sidx_v, pidx_v, srows_v, prows_v, out_v, sem):
    n_ch = b_per_w // CH
    wid = lax.axis_index("s") * NC + lax.axis_index("c")
    base = pl.multiple_of(wid * b_per_w, b_per_w)

    for c in range(n_ch):
        pltpu.sync_copy(sid_hbm.at[pl.ds(base + c * CH, CH)], sidx_v.at[c])
        pltpu.sync_copy(pid_hbm.at[pl.ds(base + c * CH, CH)], pidx_v.at[c])

    copies = []
    for c in range(n_ch):
        copies.append(pltpu.async_copy(
            semb_hbm.at[sidx_v.at[c]], srows_v.at[pl.ds(c * CH, CH)], sem))
        copies.append(pltpu.async_copy(
            pemb_hbm.at[pidx_v.at[c]], prows_v.at[pl.ds(c * CH, CH)], sem))
    for cp in copies:
        cp.wait()

    U = 8
    lane = lax.iota(jnp.int32, L)
    last_lane = lane == (L - 1)

    def group(g, carry):
        r0 = pl.multiple_of(g * U, U)
        for u in range(U):
            r = r0 + u
            q = (srows_v[r, pl.ds(0, L)] * prows_v[r, pl.ds(0, L)]
                 + srows_v[r, pl.ds(L, L)] * prows_v[r, pl.ds(L, L)])
            cum = plsc.cumsum(q)
            plsc.store_scatter(out_v, [jnp.full((L,), r, jnp.int32)], cum,
                               mask=last_lane)
        return carry

    lax.fori_loop(0, b_per_w // U, group, 0)
    pltpu.sync_copy(out_v, out_hbm.at[pl.ds(base, b_per_w)])


def kernel(sid, pid, scientist_emb, paper_emb):
    batch = sid.shape[0]
    b_per_w = batch // NW
    mesh = plsc.VectorSubcoreMesh(core_axis_name="c", subcore_axis_name="s",
                                  num_cores=NC, num_subcores=NS)
    k = pl.kernel(
        functools.partial(_dot_body, b_per_w),
        out_type=jax.ShapeDtypeStruct((batch,), jnp.float32),
        mesh=mesh,
        scratch_types=[
            pltpu.VMEM((b_per_w // CH, CH), jnp.int32),
            pltpu.VMEM((b_per_w // CH, CH), jnp.int32),
            pltpu.VMEM((b_per_w, D), jnp.float32),
            pltpu.VMEM((b_per_w, D), jnp.float32),
            pltpu.VMEM((b_per_w,), jnp.float32),
            pltpu.SemaphoreType.DMA,
        ],
        compiler_params=pltpu.CompilerParams(needs_layout_passes=False,
                                             use_tc_tiling_on_sc=False),
    )
    return k(sid.astype(jnp.int32), pid.astype(jnp.int32),
             scientist_emb, paper_emb)


# speedup vs baseline: 1.4882x; 1.4882x over previous
import functools

import jax
import jax.numpy as jnp
from jax import lax
from jax.experimental import pallas as pl
from jax.experimental.pallas import tpu as pltpu
from jax.experimental.pallas import tpu_sc as plsc

D = 32
L = 16
NC = 2
NS = 16
NW = NC * NS
CH = 128


def _dot_body(b_per_w, sid_hbm, pid_hbm, semb_hbm, pemb_hbm, out_hbm,
              sidx_v, pidx_v, srows_v, prows_v, out_v, sem):
    n_ch = b_per_w // CH
    wid = lax.axis_index("s") * NC + lax.axis_index("c")
    base = pl.multiple_of(wid * b_per_w, b_per_w)

    for c in range(n_ch):
        pltpu.sync_copy(sid_hbm.at[pl.ds(base + c * CH, CH)], sidx_v.at[c])
        pltpu.sync_copy(pid_hbm.at[pl.ds(base + c * CH, CH)], pidx_v.at[c])

    lane = lax.iota(jnp.int32, L)
    last_lane = lane == (L - 1)

    def group(g, carry):
        r0 = pl.multiple_of(g * L, L)
        c = g // (CH // L)
        off = pl.multiple_of((g % (CH // L)) * L, L)
        sidx = sidx_v[c, pl.ds(off, L)]
        pidx = pidx_v[c, pl.ds(off, L)]
        copies = []
        for u in range(L):
            sr = lax.convert_element_type(sidx[u], jnp.int32)
            pr = lax.convert_element_type(pidx[u], jnp.int32)
            copies.append(pltpu.async_copy(
                semb_hbm.at[pl.ds(sr, 1)], srows_v.at[pl.ds(u, 1)], sem))
            copies.append(pltpu.async_copy(
                pemb_hbm.at[pl.ds(pr, 1)], prows_v.at[pl.ds(u, 1)], sem))
        for cp in copies:
            cp.wait()
        prods = []
        for u in range(L):
            q = (srows_v[u, pl.ds(0, L)] * prows_v[u, pl.ds(0, L)]
                 + srows_v[u, pl.ds(L, L)] * prows_v[u, pl.ds(L, L)])
            cum = plsc.cumsum(q)
            plsc.store_scatter(out_v, [jnp.full((L,), r0 + u, jnp.int32)],
                               cum, mask=last_lane)
        return carry

    lax.fori_loop(0, b_per_w // L, group, 0)
    pltpu.sync_copy(out_v, out_hbm.at[pl.ds(base, b_per_w)])


def kernel(sid, pid, scientist_emb, paper_emb):
    batch = sid.shape[0]
    b_per_w = batch // NW
    mesh = plsc.VectorSubcoreMesh(core_axis_name="c", subcore_axis_name="s",
                                  num_cores=NC, num_subcores=NS)
    k = pl.kernel(
        functools.partial(_dot_body, b_per_w),
        out_type=jax.ShapeDtypeStruct((batch,), jnp.float32),
        mesh=mesh,
        scratch_types=[
            pltpu.VMEM((b_per_w // CH, CH), jnp.int32),
            pltpu.VMEM((b_per_w // CH, CH), jnp.int32),
            pltpu.VMEM((L, D), jnp.float32),
            pltpu.VMEM((L, D), jnp.float32),
            pltpu.VMEM((b_per_w,), jnp.float32),
            pltpu.SemaphoreType.DMA,
        ],
        compiler_params=pltpu.CompilerParams(needs_layout_passes=False,
                                             use_tc_tiling_on_sc=True),
    )
    return k(sid.astype(jnp.int32), pid.astype(jnp.int32),
             scientist_emb, paper_emb)


# speedup vs baseline: 1.5863x; 1.0659x over previous
import functools

import jax
import jax.numpy as jnp
from jax import lax
from jax.experimental import pallas as pl
from jax.experimental.pallas import tpu as pltpu
from jax.experimental.pallas import tpu_sc as plsc

D = 32
L = 16
NC = 2
NS = 16
NW = NC * NS
W = 32


def _dot_body(b_per_w, sid_hbm, pid_hbm, semb_hbm, pemb_hbm, out_hbm,
              idx_v, srows_v, prows_v, out_v, ssem, psem):
    n_w = b_per_w // W
    wid = lax.axis_index("s") * NC + lax.axis_index("c")
    base = pl.multiple_of(wid * b_per_w, b_per_w)

    pltpu.sync_copy(sid_hbm.at[pl.ds(base, b_per_w)], idx_v.at[0])
    pltpu.sync_copy(pid_hbm.at[pl.ds(base, b_per_w)], idx_v.at[1])

    lane = lax.iota(jnp.int32, L)
    last_lane = lane == (L - 1)

    def fire(w, buf):
        r0 = pl.multiple_of(w * W, W)
        for g in range(W // L):
            svec = idx_v[0, pl.ds(r0 + g * L, L)]
            pvec = idx_v[1, pl.ds(r0 + g * L, L)]
            for u in range(L):
                row = g * L + u
                pltpu.async_copy(semb_hbm.at[pl.ds(svec[u], 1)],
                                 srows_v.at[buf, pl.ds(row, 1)], ssem)
                pltpu.async_copy(pemb_hbm.at[pl.ds(pvec[u], 1)],
                                 prows_v.at[buf, pl.ds(row, 1)], psem)

    def drain():
        pltpu.make_async_copy(semb_hbm.at[pl.ds(0, W)],
                              srows_v.at[0], ssem).wait()
        pltpu.make_async_copy(pemb_hbm.at[pl.ds(0, W)],
                              prows_v.at[0], psem).wait()

    def compute(w, buf):
        r0 = pl.multiple_of(w * W, W)
        for u in range(W):
            q = (srows_v[buf, u, pl.ds(0, L)] * prows_v[buf, u, pl.ds(0, L)]
                 + srows_v[buf, u, pl.ds(L, L)]
                 * prows_v[buf, u, pl.ds(L, L)])
            cum = plsc.cumsum(q)
            plsc.store_scatter(out_v, [jnp.full((L,), r0 + u, jnp.int32)],
                               cum, mask=last_lane)

    fire(0, 0)

    def step(w, carry):
        @pl.when(w < n_w - 1)
        def _():
            fire(w + 1, (w + 1) & 1)
        drain()
        compute(w, w & 1)
        return carry

    lax.fori_loop(0, n_w, step, 0)
    pltpu.sync_copy(out_v, out_hbm.at[pl.ds(base, b_per_w)])


def kernel(sid, pid, scientist_emb, paper_emb):
    batch = sid.shape[0]
    b_per_w = batch // NW
    mesh = plsc.VectorSubcoreMesh(core_axis_name="c", subcore_axis_name="s",
                                  num_cores=NC, num_subcores=NS)
    k = pl.kernel(
        functools.partial(_dot_body, b_per_w),
        out_type=jax.ShapeDtypeStruct((batch,), jnp.float32),
        mesh=mesh,
        scratch_types=[
            pltpu.VMEM((2, b_per_w), jnp.int32),
            pltpu.VMEM((2, W, D), jnp.float32),
            pltpu.VMEM((2, W, D), jnp.float32),
            pltpu.VMEM((b_per_w,), jnp.float32),
            pltpu.SemaphoreType.DMA,
            pltpu.SemaphoreType.DMA,
        ],
        compiler_params=pltpu.CompilerParams(needs_layout_passes=False,
                                             use_tc_tiling_on_sc=True),
    )
    return k(sid.astype(jnp.int32), pid.astype(jnp.int32),
             scientist_emb, paper_emb)
